# Initial kernel scaffold; baseline (speedup 1.0000x reference)
import jax
import jax.numpy as jnp
from jax.experimental import pallas as pl


def kernel(x, pe):
    raise NotImplementedError("write your pallas kernel here")


# speedup vs baseline: 4.1436x; 4.1436x over previous
import functools

import jax
import jax.numpy as jnp
from jax import lax
from jax.experimental import pallas as pl
from jax.experimental.pallas import tpu as pltpu
from jax.experimental.pallas import tpu_sc as plsc

D_MODEL = 64
NC = 2
NS = 16
NW = NC * NS
CHUNK = 1024


@functools.lru_cache(maxsize=None)
def _make_sc_gather(B, b_per_w, n_chunks):
    mesh = plsc.VectorSubcoreMesh(core_axis_name="c", subcore_axis_name="s")

    @functools.partial(
        pl.kernel,
        mesh=mesh,
        out_type=jax.ShapeDtypeStruct((B, D_MODEL), jnp.float32),
        scratch_types=[
            pltpu.VMEM((CHUNK,), jnp.int32),
            pltpu.VMEM((CHUNK, D_MODEL), jnp.float32),
            pltpu.SemaphoreType.DMA,
        ],
        compiler_params=pltpu.CompilerParams(use_tc_tiling_on_sc=False),
    )
    def k(table_hbm, idx_hbm, out_hbm, idx_v, rows_v, sem):
        wid = lax.axis_index("s") * NC + lax.axis_index("c")
        wbase = wid * b_per_w

        def body(c, carry):
            base = wbase + c * CHUNK
            pltpu.sync_copy(idx_hbm.at[pl.ds(base, CHUNK)], idx_v)
            pltpu.async_copy(table_hbm.at[idx_v], rows_v, sem).wait()
            pltpu.sync_copy(rows_v, out_hbm.at[pl.ds(base, CHUNK)])
            return carry

        lax.fori_loop(0, n_chunks, body, 0)

    return k


def kernel(x, pe):
    nb, nh = x.shape
    B = nb * nh
    idx = x.reshape(B).astype(jnp.int32)
    b_per_w = B // NW
    n_chunks = b_per_w // CHUNK
    out = _make_sc_gather(B, b_per_w, n_chunks)(pe, idx)
    return out.reshape(nb, nh, D_MODEL)


# speedup vs baseline: 4.1634x; 1.0048x over previous
import functools

import jax
import jax.numpy as jnp
from jax import lax
from jax.experimental import pallas as pl
from jax.experimental.pallas import tpu as pltpu
from jax.experimental.pallas import tpu_sc as plsc

D_MODEL = 64
NC = 2
NS = 16
NW = NC * NS
CHUNK = 800


@functools.lru_cache(maxsize=None)
def _make_sc_gather(B, b_per_w, n_chunks):
    assert n_chunks % 2 == 0 and n_chunks >= 4
    mesh = plsc.VectorSubcoreMesh(core_axis_name="c", subcore_axis_name="s")

    @functools.partial(
        pl.kernel,
        mesh=mesh,
        out_type=jax.ShapeDtypeStruct((B, D_MODEL), jnp.float32),
        scratch_types=[
            pltpu.VMEM((CHUNK,), jnp.int32),
            pltpu.VMEM((CHUNK,), jnp.int32),
            pltpu.VMEM((CHUNK, D_MODEL), jnp.float32),
            pltpu.VMEM((CHUNK, D_MODEL), jnp.float32),
            pltpu.SemaphoreType.DMA,
            pltpu.SemaphoreType.DMA,
            pltpu.SemaphoreType.DMA,
            pltpu.SemaphoreType.DMA,
            pltpu.SemaphoreType.DMA,
            pltpu.SemaphoreType.DMA,
        ],
        compiler_params=pltpu.CompilerParams(use_tc_tiling_on_sc=False),
    )
    def k(table_hbm, idx_hbm, out_hbm, idx0, idx1, rows0, rows1,
          isem0, isem1, gsem0, gsem1, osem0, osem1):
        wid = lax.axis_index("s") * NC + lax.axis_index("c")
        wbase = wid * b_per_w
        idx_v = (idx0, idx1)
        rows_v = (rows0, rows1)
        isem = (isem0, isem1)
        gsem = (gsem0, gsem1)
        osem = (osem0, osem1)
        last = n_chunks - 1

        def idx_load(c, s):
            cc = lax.min(c, last)
            pltpu.async_copy(idx_hbm.at[pl.ds(wbase + cc * CHUNK, CHUNK)],
                             idx_v[s], isem[s])

        def step(c, s, first):
            base = wbase + c * CHUNK
            pltpu.make_async_copy(idx_hbm.at[pl.ds(base, CHUNK)],
                                  idx_v[s], isem[s]).wait()
            if not first:
                pltpu.make_async_copy(rows_v[s],
                                      out_hbm.at[pl.ds(base, CHUNK)],
                                      osem[s]).wait()
            pltpu.async_copy(table_hbm.at[idx_v[s]], rows_v[s], gsem[s])
            pltpu.make_async_copy(table_hbm.at[idx_v[s]], rows_v[s],
                                  gsem[s]).wait()
            idx_load(c + 2, s)
            pltpu.async_copy(rows_v[s], out_hbm.at[pl.ds(base, CHUNK)],
                             osem[s])

        idx_load(0, 0)
        idx_load(1, 1)
        step(0, 0, True)
        step(1, 1, True)

        def body(g, carry):
            step(2 * g, 0, False)
            step(2 * g + 1, 1, False)
            return carry

        lax.fori_loop(1, n_chunks // 2, body, 0)

        for s in (0, 1):
            pltpu.make_async_copy(idx_hbm.at[pl.ds(wbase, CHUNK)],
                                  idx_v[s], isem[s]).wait()
            pltpu.make_async_copy(rows_v[s], out_hbm.at[pl.ds(wbase, CHUNK)],
                                  osem[s]).wait()

    return k


def kernel(x, pe):
    nb, nh = x.shape
    B = nb * nh
    idx = x.reshape(B).astype(jnp.int32)
    b_per_w = B // NW
    n_chunks = b_per_w // CHUNK
    out = _make_sc_gather(B, b_per_w, n_chunks)(pe, idx)
    return out.reshape(nb, nh, D_MODEL)


# speedup vs baseline: 4.1657x; 1.0006x over previous
import functools

import jax
import jax.numpy as jnp
from jax import lax
from jax.experimental import pallas as pl
from jax.experimental.pallas import tpu as pltpu
from jax.experimental.pallas import tpu_sc as plsc

D_MODEL = 64
NC = 2
NS = 16
NW = NC * NS
BAT = 4


@functools.lru_cache(maxsize=None)
def _make_sc_gather(NB, NH):
    assert NB % (NW * BAT) == 0
    bat_per_w = NB // NW
    n_chunks = bat_per_w // BAT
    CH = BAT * NH
    assert n_chunks % 2 == 0 and n_chunks >= 4
    mesh = plsc.VectorSubcoreMesh(core_axis_name="c", subcore_axis_name="s")

    @functools.partial(
        pl.kernel,
        mesh=mesh,
        out_type=jax.ShapeDtypeStruct((NB, NH, D_MODEL), jnp.float32),
        scratch_types=[
            pltpu.VMEM((CH,), jnp.int32),
            pltpu.VMEM((CH,), jnp.int32),
            pltpu.VMEM((CH, D_MODEL), jnp.float32),
            pltpu.VMEM((CH, D_MODEL), jnp.float32),
            pltpu.SemaphoreType.DMA,
            pltpu.SemaphoreType.DMA,
            pltpu.SemaphoreType.DMA,
            pltpu.SemaphoreType.DMA,
            pltpu.SemaphoreType.DMA,
            pltpu.SemaphoreType.DMA,
        ],
        compiler_params=pltpu.CompilerParams(use_tc_tiling_on_sc=False),
    )
    def k(table_hbm, idx_hbm, out_hbm, idx0, idx1, rows0, rows1,
          isem0, isem1, gsem0, gsem1, osem0, osem1):
        wid = lax.axis_index("s") * NC + lax.axis_index("c")
        wbat = wid * bat_per_w
        idx_v = (idx0, idx1)
        rows_v = (rows0, rows1)
        isem = (isem0, isem1)
        gsem = (gsem0, gsem1)
        osem = (osem0, osem1)
        last = n_chunks - 1

        def idx_load(c, s):
            cc = lax.min(c, last)
            pltpu.async_copy(
                idx_hbm.at[pl.ds((wbat + cc * BAT) * NH, CH)],
                idx_v[s], isem[s])

        def store(c, s):
            b0 = wbat + c * BAT
            for b in range(BAT):
                pltpu.async_copy(rows_v[s].at[pl.ds(b * NH, NH)],
                                 out_hbm.at[b0 + b], osem[s])

        def store_wait(c, s):
            b0 = wbat + c * BAT
            for b in range(BAT):
                pltpu.make_async_copy(rows_v[s].at[pl.ds(b * NH, NH)],
                                      out_hbm.at[b0 + b], osem[s]).wait()

        def step(c, s, first):
            pltpu.make_async_copy(
                idx_hbm.at[pl.ds((wbat + c * BAT) * NH, CH)],
                idx_v[s], isem[s]).wait()
            if not first:
                store_wait(c, s)
            pltpu.async_copy(table_hbm.at[idx_v[s]], rows_v[s],
                             gsem[s]).wait()
            idx_load(c + 2, s)
            store(c, s)

        idx_load(0, 0)
        idx_load(1, 1)
        step(0, 0, True)
        step(1, 1, True)

        def body(g, carry):
            step(2 * g, 0, False)
            step(2 * g + 1, 1, False)
            return carry

        lax.fori_loop(1, n_chunks // 2, body, 0)

        for s in (0, 1):
            pltpu.make_async_copy(idx_hbm.at[pl.ds(0, CH)],
                                  idx_v[s], isem[s]).wait()
            store_wait(0, s)

    return k


def kernel(x, pe):
    nb, nh = x.shape
    idx = x.reshape(nb * nh).astype(jnp.int32)
    return _make_sc_gather(nb, nh)(pe, idx)


# speedup vs baseline: 8.3223x; 1.9978x over previous
import functools

import jax
import jax.numpy as jnp
from jax import lax
from jax.experimental import pallas as pl
from jax.experimental.pallas import tpu as pltpu
from jax.experimental.pallas import tpu_sc as plsc

D_MODEL = 64
NC = 2
NS = 16
NW = NC * NS
CHUNK = 256
LANES = 16


@functools.lru_cache(maxsize=None)
def _make_sc_gather(B, b_per_w, n_chunks, VM):
    assert n_chunks % 2 == 0 and n_chunks >= 4
    mesh = plsc.VectorSubcoreMesh(core_axis_name="c", subcore_axis_name="s")

    @functools.partial(
        pl.kernel,
        mesh=mesh,
        out_type=jax.ShapeDtypeStruct((B, D_MODEL), jnp.float32),
        scratch_types=[
            pltpu.VMEM_SHARED((VM, 2 * D_MODEL), jnp.float32),
            pltpu.VMEM((CHUNK,), jnp.int32),
            pltpu.VMEM((CHUNK,), jnp.int32),
            pltpu.VMEM((CHUNK, 2 * D_MODEL), jnp.float32),
            pltpu.VMEM((CHUNK, 2 * D_MODEL), jnp.float32),
            pltpu.VMEM((CHUNK, D_MODEL), jnp.float32),
            pltpu.SemaphoreType.DMA,
            pltpu.SemaphoreType.DMA,
            pltpu.SemaphoreType.DMA,
            pltpu.SemaphoreType.DMA,
            pltpu.SemaphoreType.DMA,
        ],
        compiler_params=pltpu.CompilerParams(use_tc_tiling_on_sc=True),
    )
    def k(table_hbm, idx_hbm, out_hbm, table_sh, idx0, idx1, bufg0, bufg1,
          bufs, isem0, isem1, gsem0, gsem1, osem):
        sid = lax.axis_index("s")
        wid = sid * NC + lax.axis_index("c")
        wbase = wid * b_per_w
        idx_v = (idx0, idx1)
        bufg = (bufg0, bufg1)
        isem = (isem0, isem1)
        gsem = (gsem0, gsem1)
        last = n_chunks - 1

        @pl.when(sid == 0)
        def _():
            pltpu.sync_copy(table_hbm, table_sh)

        plsc.subcore_barrier()

        def idx_load(c, s):
            cc = lax.min(c, last)
            pltpu.async_copy(idx_hbm.at[pl.ds(wbase + cc * CHUNK, CHUNK)],
                             idx_v[s], isem[s])

        def idx_wait(s):
            pltpu.make_async_copy(idx_hbm.at[pl.ds(wbase, CHUNK)],
                                  idx_v[s], isem[s]).wait()

        def gather_start(s):
            pltpu.async_copy(table_sh.at[idx_v[s]], bufg[s], gsem[s])

        def gather_wait(s):
            pltpu.make_async_copy(table_sh.at[idx_v[s]], bufg[s],
                                  gsem[s]).wait()

        def vcopy(s):
            g = bufg[s]

            def body(r, carry):
                for j in range(D_MODEL // LANES):
                    bufs[r, pl.ds(j * LANES, LANES)] = (
                        g[r, pl.ds(j * LANES, LANES)])
                return carry

            lax.fori_loop(0, CHUNK, body, 0)

        def store_start(c):
            pltpu.async_copy(bufs, out_hbm.at[pl.ds(wbase + c * CHUNK,
                                                    CHUNK)], osem)

        def store_wait(c):
            pltpu.make_async_copy(bufs, out_hbm.at[pl.ds(wbase + c * CHUNK,
                                                         CHUNK)], osem).wait()

        def step(c, s, first):
            idx_wait(s ^ 1)
            idx_load(c + 2, s)
            gather_start(s ^ 1)
            if not first:
                store_wait(c - 1)
            vcopy(s)
            store_start(c)
            gather_wait(s ^ 1)

        idx_load(0, 0)
        idx_load(1, 1)
        idx_wait(0)
        gather_start(0)
        gather_wait(0)
        step(0, 0, True)

        def body(g, carry):
            step(2 * g + 1, 1, False)
            step(2 * g + 2, 0, False)
            return carry

        lax.fori_loop(0, (n_chunks - 2) // 2, body, 0)

        step(n_chunks - 1, 1, False)
        store_wait(n_chunks - 1)
        idx_wait(1)

    return k


def kernel(x, pe):
    nb, nh = x.shape
    B = nb * nh
    V = pe.shape[0]
    H = V // 2
    idx = x.reshape(B).astype(jnp.int32)
    pe_sh = jnp.concatenate([pe[1:], jnp.zeros((1, D_MODEL), jnp.float32)], 0)
    table_m = jnp.concatenate([pe.reshape(H, 2 * D_MODEL),
                               pe_sh.reshape(H, 2 * D_MODEL)], 0)
    r = (idx >> 1) + (idx & 1) * H
    b_per_w = B // NW
    n_chunks = b_per_w // CHUNK
    out = _make_sc_gather(B, b_per_w, n_chunks, V)(table_m, r)
    return out.reshape(nb, nh, D_MODEL)
